# baseline (speedup 1.0000x reference)
import functools

import jax
import jax.numpy as jnp
from jax import lax
from jax.experimental import pallas as pl
from jax.experimental.pallas import tpu as pltpu
from jax.experimental.pallas import tpu_sc as plsc

B = 16
MAX_LEN = 4096
TOTAL = B * (MAX_LEN // 2)
FLAT = TOTAL * 4
ROW_W = MAX_LEN * 4
QTR_W = ROW_W // 4
WINQ = 9 * 512
W0Q_MAX = FLAT - WINQ
NU = QTR_W // 16


def _body(xin_hbm, cu_hbm, out_hbm, cu_v, in1_v, in2_v, out_v,
          sem1, sem2, semo):
    cid = lax.axis_index("c")
    sid = lax.axis_index("s")
    t = sid * 2 + cid
    b1 = t // 4
    q1 = lax.rem(t, 4)
    b2 = b1 + 8
    q2 = 3 - q1

    pltpu.sync_copy(cu_hbm, cu_v.at[pl.ds(0, B + 1)])

    lanes = lax.iota(jnp.int32, 16)
    neg1 = jnp.full((16,), -1.0, jnp.float32)

    def job_scalars(b, q):
        s = cu_v[pl.ds(b, 16)][0]
        e = cu_v[pl.ds(b + 1, 16)][0]
        length = e - s
        m = lax.rem(s, 128)
        gq0 = q * 8
        w0 = jnp.minimum((lax.div(s, 128) + gq0) * 512, W0Q_MAX)
        dd = ((lax.div(s, 128) + gq0) - lax.div(w0, 512)) * 512 + m
        lv = jnp.clip(length - q * 1024, 0, 1024)
        i1 = (lv // 128) * 32
        i2 = ((lv + 127) // 128) * 32
        return s, length, m, gq0, w0, dd, i1, i2

    s1, len1, m1, gq01, w01, dd1, i1a, i2a = job_scalars(b1, q1)
    s2, len2, m2, gq02, w02, dd2, i1b, i2b = job_scalars(b2, q2)

    cp1 = pltpu.async_copy(
        xin_hbm.at[pl.ds(w01, WINQ)], in1_v.at[pl.ds(0, WINQ)], sem1)
    cp2 = pltpu.async_copy(
        xin_hbm.at[pl.ds(w02, WINQ)], in2_v.at[pl.ds(0, WINQ)], sem2)

    def make_units(in_ref, dd, m, length, gq0, jobbase):
        def load_unit(i):
            off = i * 16 + dd
            p1 = jnp.minimum(off, WINQ)
            p2 = jnp.minimum(off + 384, WINQ)
            x1 = in_ref[pl.ds(p1, 16)]
            x2 = in_ref[pl.ds(p2, 16)]
            kpos = (i % 8) * 16 + lanes
            return kpos, jnp.where(kpos < 128 - m, x1, x2)

        def unit_full(i, carry=None):
            _, x = load_unit(i)
            out_v[pl.ds(jobbase + i * 16, 16)] = x

        def unit_edge(i, carry=None):
            kpos, x = load_unit(i)
            jvec = (gq0 + i // 32) * 128 + kpos
            out_v[pl.ds(jobbase + i * 16, 16)] = jnp.where(
                jvec < length, x, -1.0)

        def unit_pad(i, carry=None):
            out_v[pl.ds(jobbase + i * 16, 16)] = neg1

        return unit_full, unit_edge, unit_pad

    full1, edge1, pad1 = make_units(in1_v, dd1, m1, len1, gq01, 0)
    full2, edge2, pad2 = make_units(in2_v, dd2, m2, len2, gq02, QTR_W)

    plsc.parallel_loop(i2a, NU, unroll=8)(pad1)
    plsc.parallel_loop(i2b, NU, unroll=8)(pad2)

    cp1.wait()
    plsc.parallel_loop(0, i1a, unroll=8)(full1)
    lax.fori_loop(i1a, i2a, edge1, None)
    cpo = pltpu.async_copy(
        out_v.at[pl.ds(0, QTR_W)],
        out_hbm.at[pl.ds(b1 * ROW_W + q1 * QTR_W, QTR_W)], semo)

    cp2.wait()
    plsc.parallel_loop(0, i1b, unroll=8)(full2)
    lax.fori_loop(i1b, i2b, edge2, None)
    pltpu.sync_copy(
        out_v.at[pl.ds(QTR_W, QTR_W)],
        out_hbm.at[pl.ds(b2 * ROW_W + q2 * QTR_W, QTR_W)])
    cpo.wait()


@jax.jit
def _bbox_to_dense(xin, cu):
    mesh = plsc.VectorSubcoreMesh(core_axis_name="c", subcore_axis_name="s")
    run = functools.partial(
        pl.kernel,
        out_type=jax.ShapeDtypeStruct((B * ROW_W,), jnp.float32),
        mesh=mesh,
        scratch_types=[
            pltpu.VMEM((32,), jnp.int32),
            pltpu.VMEM((WINQ + 16,), jnp.float32),
            pltpu.VMEM((WINQ + 16,), jnp.float32),
            pltpu.VMEM((2 * QTR_W,), jnp.float32),
            pltpu.SemaphoreType.DMA,
            pltpu.SemaphoreType.DMA,
            pltpu.SemaphoreType.DMA,
        ],
    )(_body)
    return run(xin, cu)


def kernel(bbox_values, cu_seqlens, keep_ragged):
    xin = bbox_values.reshape(256, 128, 4).transpose(0, 2, 1).reshape(-1)
    out = _bbox_to_dense(xin, cu_seqlens.astype(jnp.int32))
    return out.reshape(B, 32, 4, 128).transpose(0, 1, 3, 2).reshape(B, MAX_LEN, 4)


# speedup vs baseline: 1.0276x; 1.0276x over previous
import functools

import jax
import jax.numpy as jnp
from jax import lax
from jax.experimental import pallas as pl
from jax.experimental.pallas import tpu as pltpu
from jax.experimental.pallas import tpu_sc as plsc

B = 16
MAX_LEN = 4096
TOTAL = B * (MAX_LEN // 2)
FLAT = TOTAL * 4
ROW_W = MAX_LEN * 4
QTR_W = ROW_W // 4
WINQ = 9 * 512
W0Q_MAX = FLAT - WINQ
NU = QTR_W // 16


def _body(xin_hbm, cu_hbm, out_hbm, cu_v, in1_v, in2_v, out_v,
          sem1, sem2, semo):
    cid = lax.axis_index("c")
    sid = lax.axis_index("s")
    t = sid * 2 + cid
    b1 = t // 4
    q1 = lax.rem(t, 4)
    b2 = b1 + 8
    q2 = 3 - q1

    pltpu.sync_copy(cu_hbm, cu_v.at[pl.ds(0, B + 1)])

    lanes = lax.iota(jnp.int32, 16)
    neg1 = jnp.full((16,), -1.0, jnp.float32)

    def job_scalars(b, q):
        s = cu_v[pl.ds(b, 16)][0]
        e = cu_v[pl.ds(b + 1, 16)][0]
        length = e - s
        m = lax.rem(s, 128)
        gq0 = q * 8
        w0 = jnp.minimum((lax.div(s, 128) + gq0) * 512, W0Q_MAX)
        dd = ((lax.div(s, 128) + gq0) - lax.div(w0, 512)) * 512 + m
        lv = jnp.clip(length - q * 1024, 0, 1024)
        i1 = (lv // 128) * 32
        i2 = ((lv + 127) // 128) * 32
        return s, length, m, gq0, w0, dd, i1, i2

    s1, len1, m1, gq01, w01, dd1, i1a, i2a = job_scalars(b1, q1)
    s2, len2, m2, gq02, w02, dd2, i1b, i2b = job_scalars(b2, q2)

    cp1 = pltpu.async_copy(
        xin_hbm.at[pl.ds(w01, WINQ)], in1_v.at[pl.ds(0, WINQ)], sem1)
    cp2 = pltpu.async_copy(
        xin_hbm.at[pl.ds(w02, WINQ)], in2_v.at[pl.ds(0, WINQ)], sem2)

    def make_units(in_ref, dd, m, length, gq0, jobbase):
        def load_unit(i):
            off = i * 16 + dd
            p1 = jnp.minimum(off, WINQ)
            p2 = jnp.minimum(off + 384, WINQ)
            x1 = in_ref[pl.ds(p1, 16)]
            x2 = in_ref[pl.ds(p2, 16)]
            kpos = (i % 8) * 16 + lanes
            return kpos, jnp.where(kpos < 128 - m, x1, x2)

        def unit_full(i, carry=None):
            _, x = load_unit(i)
            out_v[pl.ds(jobbase + i * 16, 16)] = x

        def unit_edge(i, carry=None):
            kpos, x = load_unit(i)
            jvec = (gq0 + i // 32) * 128 + kpos
            out_v[pl.ds(jobbase + i * 16, 16)] = jnp.where(
                jvec < length, x, -1.0)

        def unit_pad(i, carry=None):
            out_v[pl.ds(jobbase + i * 16, 16)] = neg1

        return unit_full, unit_edge, unit_pad

    full1, edge1, pad1 = make_units(in1_v, dd1, m1, len1, gq01, 0)
    full2, edge2, pad2 = make_units(in2_v, dd2, m2, len2, gq02, QTR_W)

    plsc.parallel_loop(i2a, NU, unroll=4)(pad1)
    plsc.parallel_loop(i2b, NU, unroll=4)(pad2)

    cp1.wait()
    plsc.parallel_loop(0, i1a, unroll=4)(full1)
    lax.fori_loop(i1a, i2a, edge1, None)
    cpo = pltpu.async_copy(
        out_v.at[pl.ds(0, QTR_W)],
        out_hbm.at[pl.ds(b1 * ROW_W + q1 * QTR_W, QTR_W)], semo)

    cp2.wait()
    plsc.parallel_loop(0, i1b, unroll=4)(full2)
    lax.fori_loop(i1b, i2b, edge2, None)
    pltpu.sync_copy(
        out_v.at[pl.ds(QTR_W, QTR_W)],
        out_hbm.at[pl.ds(b2 * ROW_W + q2 * QTR_W, QTR_W)])
    cpo.wait()


@jax.jit
def _bbox_to_dense(xin, cu):
    mesh = plsc.VectorSubcoreMesh(core_axis_name="c", subcore_axis_name="s")
    run = functools.partial(
        pl.kernel,
        out_type=jax.ShapeDtypeStruct((B * ROW_W,), jnp.float32),
        mesh=mesh,
        scratch_types=[
            pltpu.VMEM((32,), jnp.int32),
            pltpu.VMEM((WINQ + 16,), jnp.float32),
            pltpu.VMEM((WINQ + 16,), jnp.float32),
            pltpu.VMEM((2 * QTR_W,), jnp.float32),
            pltpu.SemaphoreType.DMA,
            pltpu.SemaphoreType.DMA,
            pltpu.SemaphoreType.DMA,
        ],
    )(_body)
    return run(xin, cu)


def kernel(bbox_values, cu_seqlens, keep_ragged):
    xin = bbox_values.reshape(256, 128, 4).transpose(0, 2, 1).reshape(-1)
    out = _bbox_to_dense(xin, cu_seqlens.astype(jnp.int32))
    return out.reshape(B, 32, 4, 128).transpose(0, 1, 3, 2).reshape(B, MAX_LEN, 4)


# speedup vs baseline: 1.0306x; 1.0029x over previous
import functools

import jax
import jax.numpy as jnp
from jax import lax
from jax.experimental import pallas as pl
from jax.experimental.pallas import tpu as pltpu
from jax.experimental.pallas import tpu_sc as plsc

B = 16
MAX_LEN = 4096
TOTAL = B * (MAX_LEN // 2)
FLAT = TOTAL * 4
ROW_W = MAX_LEN * 4
QTR_W = ROW_W // 4
WINQ = 9 * 512
W0Q_MAX = FLAT - WINQ
NU = QTR_W // 16


def _body(xin_hbm, cu_hbm, out_hbm, cu_v, in1_v, in2_v, out_v,
          sem1, sem2, semo):
    cid = lax.axis_index("c")
    sid = lax.axis_index("s")
    t = sid * 2 + cid
    b1 = t // 4
    q1 = lax.rem(t, 4)
    b2 = b1 + 8
    q2 = 3 - q1

    pltpu.sync_copy(cu_hbm, cu_v.at[pl.ds(0, B + 1)])

    lanes = lax.iota(jnp.int32, 16)
    neg1 = jnp.full((16,), -1.0, jnp.float32)

    def job_scalars(b, q):
        s = cu_v[pl.ds(b, 16)][0]
        e = cu_v[pl.ds(b + 1, 16)][0]
        length = e - s
        m = lax.rem(s, 128)
        gq0 = q * 8
        w0 = jnp.minimum((lax.div(s, 128) + gq0) * 512, W0Q_MAX)
        dd = ((lax.div(s, 128) + gq0) - lax.div(w0, 512)) * 512 + m
        lv = jnp.clip(length - q * 1024, 0, 1024)
        i1 = (lv // 128) * 32
        i2 = ((lv + 127) // 128) * 32
        return s, length, m, gq0, w0, dd, i1, i2

    s1, len1, m1, gq01, w01, dd1, i1a, i2a = job_scalars(b1, q1)
    s2, len2, m2, gq02, w02, dd2, i1b, i2b = job_scalars(b2, q2)

    cp1 = pltpu.async_copy(
        xin_hbm.at[pl.ds(w01, WINQ)], in1_v.at[pl.ds(0, WINQ)], sem1)
    cp2 = pltpu.async_copy(
        xin_hbm.at[pl.ds(w02, WINQ)], in2_v.at[pl.ds(0, WINQ)], sem2)

    def make_units(in_ref, dd, m, length, gq0, jobbase):
        def load_unit(i):
            off = i * 16 + dd
            p1 = jnp.minimum(off, WINQ)
            p2 = jnp.minimum(off + 384, WINQ)
            x1 = in_ref[pl.ds(p1, 16)]
            x2 = in_ref[pl.ds(p2, 16)]
            kpos = (i % 8) * 16 + lanes
            return kpos, jnp.where(kpos < 128 - m, x1, x2)

        def unit_full(i, carry=None):
            _, x = load_unit(i)
            out_v[pl.ds(jobbase + i * 16, 16)] = x

        def unit_edge(i, carry=None):
            kpos, x = load_unit(i)
            jvec = (gq0 + i // 32) * 128 + kpos
            out_v[pl.ds(jobbase + i * 16, 16)] = jnp.where(
                jvec < length, x, -1.0)

        def unit_pad(i, carry=None):
            out_v[pl.ds(jobbase + i * 16, 16)] = neg1

        return unit_full, unit_edge, unit_pad

    full1, edge1, pad1 = make_units(in1_v, dd1, m1, len1, gq01, 0)
    full2, edge2, pad2 = make_units(in2_v, dd2, m2, len2, gq02, QTR_W)

    plsc.parallel_loop(i2a, NU, unroll=4)(pad1)
    plsc.parallel_loop(i2b, NU, unroll=4)(pad2)

    cp1.wait()
    plsc.parallel_loop(0, i1a, unroll=4)(full1)
    plsc.parallel_loop(i1a, i2a)(edge1)
    cpo = pltpu.async_copy(
        out_v.at[pl.ds(0, QTR_W)],
        out_hbm.at[pl.ds(b1 * ROW_W + q1 * QTR_W, QTR_W)], semo)

    cp2.wait()
    plsc.parallel_loop(0, i1b, unroll=4)(full2)
    plsc.parallel_loop(i1b, i2b)(edge2)
    pltpu.sync_copy(
        out_v.at[pl.ds(QTR_W, QTR_W)],
        out_hbm.at[pl.ds(b2 * ROW_W + q2 * QTR_W, QTR_W)])
    cpo.wait()


@jax.jit
def _bbox_to_dense(xin, cu):
    mesh = plsc.VectorSubcoreMesh(core_axis_name="c", subcore_axis_name="s")
    run = functools.partial(
        pl.kernel,
        out_type=jax.ShapeDtypeStruct((B * ROW_W,), jnp.float32),
        mesh=mesh,
        scratch_types=[
            pltpu.VMEM((32,), jnp.int32),
            pltpu.VMEM((WINQ + 16,), jnp.float32),
            pltpu.VMEM((WINQ + 16,), jnp.float32),
            pltpu.VMEM((2 * QTR_W,), jnp.float32),
            pltpu.SemaphoreType.DMA,
            pltpu.SemaphoreType.DMA,
            pltpu.SemaphoreType.DMA,
        ],
    )(_body)
    return run(xin, cu)


def kernel(bbox_values, cu_seqlens, keep_ragged):
    xin = bbox_values.reshape(256, 128, 4).transpose(0, 2, 1).reshape(-1)
    out = _bbox_to_dense(xin, cu_seqlens.astype(jnp.int32))
    return out.reshape(B, 32, 4, 128).transpose(0, 1, 3, 2).reshape(B, MAX_LEN, 4)
